# baseline (speedup 1.0000x reference)
import functools

import jax
import jax.numpy as jnp
from jax import lax
from jax.experimental import pallas as pl
from jax.experimental.pallas import tpu as pltpu
from jax.experimental.pallas import tpu_sc as plsc

F32 = jnp.float32
_PREC = lax.Precision.HIGHEST

NC = 2
NS = 16
NW = NC * NS
CHUNK = 40


def _mesh():
    return plsc.VectorSubcoreMesh(
        core_axis_name="c", subcore_axis_name="s", num_cores=NC, num_subcores=NS
    )


@functools.lru_cache(maxsize=None)
def _make_proj(N, D, H, MB):
    def body(x_ref, ws_ref, wd_ref, ps_ref, pd_ref):
        xv = x_ref[...]
        ps_ref[...] = jnp.dot(xv, ws_ref[...], precision=_PREC, preferred_element_type=F32)
        pd_ref[...] = jnp.dot(xv, wd_ref[...], precision=_PREC, preferred_element_type=F32)

    return pl.pallas_call(
        body,
        grid=(N // MB,),
        in_specs=[
            pl.BlockSpec((MB, D), lambda i: (i, 0)),
            pl.BlockSpec((D, H), lambda i: (0, 0)),
            pl.BlockSpec((D, H), lambda i: (0, 0)),
        ],
        out_specs=[
            pl.BlockSpec((MB, H), lambda i: (i, 0)),
            pl.BlockSpec((MB, H), lambda i: (i, 0)),
        ],
        out_shape=[
            jax.ShapeDtypeStruct((N, H), F32),
            jax.ShapeDtypeStruct((N, H), F32),
        ],
    )


@functools.lru_cache(maxsize=None)
def _make_edge_tail(E, DE, H, EB):
    def body(hp_ref, ea_ref, wea_ref, b1_ref, we2_ref, b2_ref, out_ref):
        h = hp_ref[...] + jnp.dot(
            ea_ref[...], wea_ref[...], precision=_PREC, preferred_element_type=F32
        ) + b1_ref[...]
        h = jnp.maximum(h, 0.0)
        out_ref[...] = jnp.dot(
            h, we2_ref[...], precision=_PREC, preferred_element_type=F32
        ) + b2_ref[...]

    return pl.pallas_call(
        body,
        grid=(E // EB,),
        in_specs=[
            pl.BlockSpec((EB, H), lambda i: (i, 0)),
            pl.BlockSpec((EB, DE), lambda i: (i, 0)),
            pl.BlockSpec((DE, H), lambda i: (0, 0)),
            pl.BlockSpec((1, H), lambda i: (0, 0)),
            pl.BlockSpec((H, DE), lambda i: (0, 0)),
            pl.BlockSpec((1, DE), lambda i: (0, 0)),
        ],
        out_specs=pl.BlockSpec((EB, DE), lambda i: (i, 0)),
        out_shape=jax.ShapeDtypeStruct((E, DE), F32),
    )


@functools.lru_cache(maxsize=None)
def _make_node(N, NP, D, DE, H, MB):
    def body(x_ref, agg_ref, w1x_ref, w1a_ref, b1_ref, w2_ref, b2_ref, out_ref):
        agg = agg_ref[0] + agg_ref[1]
        h = (
            jnp.dot(x_ref[...], w1x_ref[...], precision=_PREC, preferred_element_type=F32)
            + jnp.dot(agg, w1a_ref[...], precision=_PREC, preferred_element_type=F32)
            + b1_ref[...]
        )
        h = jnp.maximum(h, 0.0)
        out_ref[...] = jnp.dot(
            h, w2_ref[...], precision=_PREC, preferred_element_type=F32
        ) + b2_ref[...]

    return pl.pallas_call(
        body,
        grid=(N // MB,),
        in_specs=[
            pl.BlockSpec((MB, D), lambda i: (i, 0)),
            pl.BlockSpec((NC, MB, DE), lambda i: (0, i, 0)),
            pl.BlockSpec((D, H), lambda i: (0, 0)),
            pl.BlockSpec((DE, H), lambda i: (0, 0)),
            pl.BlockSpec((1, H), lambda i: (0, 0)),
            pl.BlockSpec((H, D), lambda i: (0, 0)),
            pl.BlockSpec((1, D), lambda i: (0, 0)),
        ],
        out_specs=pl.BlockSpec((MB, D), lambda i: (i, 0)),
        out_shape=jax.ShapeDtypeStruct((N, D), F32),
    )


@functools.lru_cache(maxsize=None)
def _make_sc_gather(E, N, H, EW, NCHUNK):
    assert NCHUNK % 2 == 1

    @functools.partial(
        pl.kernel,
        out_type=jax.ShapeDtypeStruct((E, H), F32),
        mesh=_mesh(),
        scratch_types=[
            pltpu.VMEM((NCHUNK, CHUNK), jnp.int32),
            pltpu.VMEM((NCHUNK, CHUNK), jnp.int32),
            pltpu.VMEM((CHUNK, H), F32),
            pltpu.VMEM((CHUNK, H), F32),
            pltpu.VMEM((CHUNK, H), F32),
            pltpu.VMEM((CHUNK, H), F32),
            pltpu.SemaphoreType.DMA,
            pltpu.SemaphoreType.DMA,
            pltpu.SemaphoreType.DMA,
            pltpu.SemaphoreType.DMA,
        ],
    )
    def body(psrc, pdst, src3, dst3, out, sidx, didx, bufa0, bufb0, bufa1, bufb1, sa0, sb0, sa1, sb1):
        c = lax.axis_index("c")
        s = lax.axis_index("s")
        wid = s * NC + c
        pltpu.sync_copy(src3.at[wid], sidx)
        pltpu.sync_copy(dst3.at[wid], didx)
        base = wid * EW

        def gather(j, bufa, bufb, sema, semb):
            return (
                pltpu.async_copy(psrc.at[sidx.at[j]], bufa, sema),
                pltpu.async_copy(pdst.at[didx.at[j]], bufb, semb),
            )

        def consume(j, bufa, bufb, sema, semb):
            pltpu.make_async_copy(psrc.at[sidx.at[j]], bufa, sema).wait()
            pltpu.make_async_copy(pdst.at[didx.at[j]], bufb, semb).wait()

            def row_body(r, carry2):
                for k in range(H // 16):
                    sl = pl.ds(k * 16, 16)
                    bufa[r, sl] = bufa[r, sl] + bufb[r, sl]
                return carry2

            lax.fori_loop(0, CHUNK, row_body, 0, unroll=4)
            pltpu.sync_copy(bufa, out.at[pl.ds(base + j * CHUNK, CHUNK)])

        gather(0, bufa0, bufb0, sa0, sb0)

        def pair_body(p, carry):
            gather(2 * p + 1, bufa1, bufb1, sa1, sb1)
            consume(2 * p, bufa0, bufb0, sa0, sb0)
            gather(2 * p + 2, bufa0, bufb0, sa0, sb0)
            consume(2 * p + 1, bufa1, bufb1, sa1, sb1)
            return carry

        lax.fori_loop(0, (NCHUNK - 1) // 2, pair_body, 0, unroll=False)
        consume(NCHUNK - 1, bufa0, bufb0, sa0, sb0)

    return body


@functools.lru_cache(maxsize=None)
def _make_sc_scatter(E, NP, DE, EW, KCH):
    FW = EW * DE
    NLOAD = FW // (KCH * 128)
    RW = (NP // NS) * DE

    @functools.partial(
        pl.kernel,
        out_type=jax.ShapeDtypeStruct((NC, NP * DE), F32),
        mesh=_mesh(),
        scratch_types=[
            pltpu.VMEM((KCH, 128), jnp.int32),
            pltpu.VMEM((KCH, 128), F32),
            pltpu.VMEM((RW,), F32),
            pltpu.VMEM_SHARED((NP * DE,), F32),
        ],
    )
    def body(msg3, fidx3, out, ibuf, mbuf, zbuf, agg_sh):
        c = lax.axis_index("c")
        s = lax.axis_index("s")
        wid = s * NC + c

        def zb(r, carry):
            zbuf[pl.ds(r * 16, 16)] = jnp.zeros((16,), F32)
            return carry

        lax.fori_loop(0, RW // 16, zb, 0, unroll=False)
        pltpu.sync_copy(zbuf, agg_sh.at[pl.ds(s * RW, RW)])
        plsc.subcore_barrier()

        def chunk(g, carry):
            pltpu.sync_copy(fidx3.at[wid * NLOAD + g], ibuf)
            pltpu.sync_copy(msg3.at[wid * NLOAD + g], mbuf)
            for k in range(KCH):
                pltpu.sync_copy(mbuf.at[k], agg_sh.at[ibuf.at[k]], add=True)
            return carry

        lax.fori_loop(0, NLOAD, chunk, 0, unroll=False)
        plsc.subcore_barrier()
        pltpu.sync_copy(agg_sh.at[pl.ds(s * RW, RW)], zbuf)
        pltpu.sync_copy(zbuf, out.at[c, pl.ds(s * RW, RW)])

    return body


def kernel(x, edge_index, edge_attr, num_nodes, W_e1, b_e1, W_e2, b_e2, W_n1, b_n1, W_n2, b_n2):
    del num_nodes
    N, D = x.shape
    E, DE = edge_attr.shape
    H = W_e1.shape[1]
    EW = E // NW
    NCHUNK = EW // CHUNK

    src = edge_index[0].astype(jnp.int32)
    dst = edge_index[1].astype(jnp.int32)
    src3 = src.reshape(NW, NCHUNK, CHUNK)
    dst3 = dst.reshape(NW, NCHUNK, CHUNK)

    RPT = -(-x.shape[0] // NS)
    RPT = -(-RPT // 128) * 128

    KCH = 25
    NLOAD = EW * DE // (KCH * 128)
    fidx3 = (dst[:, None] * DE + jnp.arange(DE, dtype=jnp.int32)[None, :]).reshape(
        NW * NLOAD, KCH, 128
    )

    W_src = W_e1[:D]
    W_dst = W_e1[D : 2 * D]
    W_ea = W_e1[2 * D :]
    W_n1x = W_n1[:D]
    W_n1a = W_n1[D:]
    b_e1r = b_e1.reshape(1, H)
    b_e2r = b_e2.reshape(1, DE)
    b_n1r = b_n1.reshape(1, H)
    b_n2r = b_n2.reshape(1, D)

    NP = RPT * NS

    proj = _make_proj(N, D, H, 2000)
    edge_tail = _make_edge_tail(E, DE, H, 4000)
    node = _make_node(N, NP, D, DE, H, 2000)
    sc_gather = _make_sc_gather(E, N, H, EW, NCHUNK)
    sc_scatter = _make_sc_scatter(E, NP, DE, EW, KCH)

    xcur = x
    msg = None
    for step in range(3):
        psrc, pdst = proj(xcur, W_src, W_dst)
        hpre = sc_gather(psrc, pdst, src3, dst3)
        msg = edge_tail(hpre, edge_attr, W_ea, b_e1r, W_e2, b_e2r)
        if step < 2:
            msg3 = msg.reshape(NW * NLOAD, KCH, 128)
            aggp = sc_scatter(msg3, fidx3).reshape(NC, NP, DE)
            xcur = node(xcur, aggp, W_n1x, W_n1a, b_n1r, W_n2, b_n2r)
    return (xcur, msg)


# speedup vs baseline: 1.1581x; 1.1581x over previous
import functools

import jax
import jax.numpy as jnp
from jax import lax
from jax.experimental import pallas as pl
from jax.experimental.pallas import tpu as pltpu
from jax.experimental.pallas import tpu_sc as plsc

F32 = jnp.float32
_PREC = lax.Precision.HIGHEST

NC = 2
NS = 16
NW = NC * NS
CHUNK = 40


def _mesh():
    return plsc.VectorSubcoreMesh(
        core_axis_name="c", subcore_axis_name="s", num_cores=NC, num_subcores=NS
    )


@functools.lru_cache(maxsize=None)
def _make_proj(N, D, H, MB):
    def body(x_ref, ws_ref, wd_ref, ps_ref, pd_ref):
        xv = x_ref[...]
        ps_ref[...] = jnp.dot(xv, ws_ref[...], precision=_PREC, preferred_element_type=F32)
        pd_ref[...] = jnp.dot(xv, wd_ref[...], precision=_PREC, preferred_element_type=F32)

    return pl.pallas_call(
        body,
        grid=(N // MB,),
        in_specs=[
            pl.BlockSpec((MB, D), lambda i: (i, 0)),
            pl.BlockSpec((D, H), lambda i: (0, 0)),
            pl.BlockSpec((D, H), lambda i: (0, 0)),
        ],
        out_specs=[
            pl.BlockSpec((MB, H), lambda i: (i, 0)),
            pl.BlockSpec((MB, H), lambda i: (i, 0)),
        ],
        out_shape=[
            jax.ShapeDtypeStruct((N, H), F32),
            jax.ShapeDtypeStruct((N, H), F32),
        ],
    )


@functools.lru_cache(maxsize=None)
def _make_edge_tail(E, DE, H, EB):
    def body(hp_ref, ea_ref, wea_ref, b1_ref, we2_ref, b2_ref, out_ref):
        h = hp_ref[...] + jnp.dot(
            ea_ref[...], wea_ref[...], precision=_PREC, preferred_element_type=F32
        ) + b1_ref[...]
        h = jnp.maximum(h, 0.0)
        out_ref[...] = jnp.dot(
            h, we2_ref[...], precision=_PREC, preferred_element_type=F32
        ) + b2_ref[...]

    return pl.pallas_call(
        body,
        grid=(E // EB,),
        in_specs=[
            pl.BlockSpec((EB, H), lambda i: (i, 0)),
            pl.BlockSpec((EB, DE), lambda i: (i, 0)),
            pl.BlockSpec((DE, H), lambda i: (0, 0)),
            pl.BlockSpec((1, H), lambda i: (0, 0)),
            pl.BlockSpec((H, DE), lambda i: (0, 0)),
            pl.BlockSpec((1, DE), lambda i: (0, 0)),
        ],
        out_specs=pl.BlockSpec((EB, DE), lambda i: (i, 0)),
        out_shape=jax.ShapeDtypeStruct((E, DE), F32),
    )


@functools.lru_cache(maxsize=None)
def _make_node(N, NP, D, DE, H, MB):
    def body(x_ref, agg_ref, w1x_ref, w1a_ref, b1_ref, w2_ref, b2_ref, out_ref):
        agg = agg_ref[0] + agg_ref[1]
        h = (
            jnp.dot(x_ref[...], w1x_ref[...], precision=_PREC, preferred_element_type=F32)
            + jnp.dot(agg, w1a_ref[...], precision=_PREC, preferred_element_type=F32)
            + b1_ref[...]
        )
        h = jnp.maximum(h, 0.0)
        out_ref[...] = jnp.dot(
            h, w2_ref[...], precision=_PREC, preferred_element_type=F32
        ) + b2_ref[...]

    return pl.pallas_call(
        body,
        grid=(N // MB,),
        in_specs=[
            pl.BlockSpec((MB, D), lambda i: (i, 0)),
            pl.BlockSpec((NC, MB, DE), lambda i: (0, i, 0)),
            pl.BlockSpec((D, H), lambda i: (0, 0)),
            pl.BlockSpec((DE, H), lambda i: (0, 0)),
            pl.BlockSpec((1, H), lambda i: (0, 0)),
            pl.BlockSpec((H, D), lambda i: (0, 0)),
            pl.BlockSpec((1, D), lambda i: (0, 0)),
        ],
        out_specs=pl.BlockSpec((MB, D), lambda i: (i, 0)),
        out_shape=jax.ShapeDtypeStruct((N, D), F32),
    )


@functools.lru_cache(maxsize=None)
def _make_sc_gather(E, N, H, EW, NCHUNK):
    assert NCHUNK % 2 == 1

    @functools.partial(
        pl.kernel,
        out_type=jax.ShapeDtypeStruct((E, H), F32),
        mesh=_mesh(),
        scratch_types=[
            pltpu.VMEM((NCHUNK, CHUNK), jnp.int32),
            pltpu.VMEM((NCHUNK, CHUNK), jnp.int32),
            pltpu.VMEM((CHUNK, H), F32),
            pltpu.VMEM((CHUNK, H), F32),
            pltpu.VMEM((CHUNK, H), F32),
            pltpu.VMEM((CHUNK, H), F32),
            pltpu.SemaphoreType.DMA,
            pltpu.SemaphoreType.DMA,
            pltpu.SemaphoreType.DMA,
            pltpu.SemaphoreType.DMA,
        ],
    )
    def body(psrc, pdst, src3, dst3, out, sidx, didx, bufa0, bufb0, bufa1, bufb1, sa0, sb0, sa1, sb1):
        c = lax.axis_index("c")
        s = lax.axis_index("s")
        wid = s * NC + c
        pltpu.sync_copy(src3.at[wid], sidx)
        pltpu.sync_copy(dst3.at[wid], didx)
        base = wid * EW

        def gather(j, bufa, bufb, sema, semb):
            return (
                pltpu.async_copy(psrc.at[sidx.at[j]], bufa, sema),
                pltpu.async_copy(pdst.at[didx.at[j]], bufb, semb),
            )

        def consume(j, bufa, bufb, sema, semb):
            pltpu.make_async_copy(psrc.at[sidx.at[j]], bufa, sema).wait()
            pltpu.make_async_copy(pdst.at[didx.at[j]], bufb, semb).wait()

            def row_body(r, carry2):
                for k in range(H // 16):
                    sl = pl.ds(k * 16, 16)
                    bufa[r, sl] = bufa[r, sl] + bufb[r, sl]
                return carry2

            lax.fori_loop(0, CHUNK, row_body, 0, unroll=False)
            pltpu.sync_copy(bufa, out.at[pl.ds(base + j * CHUNK, CHUNK)])

        gather(0, bufa0, bufb0, sa0, sb0)

        def pair_body(p, carry):
            gather(2 * p + 1, bufa1, bufb1, sa1, sb1)
            consume(2 * p, bufa0, bufb0, sa0, sb0)
            gather(2 * p + 2, bufa0, bufb0, sa0, sb0)
            consume(2 * p + 1, bufa1, bufb1, sa1, sb1)
            return carry

        lax.fori_loop(0, (NCHUNK - 1) // 2, pair_body, 0, unroll=False)
        consume(NCHUNK - 1, bufa0, bufb0, sa0, sb0)

    return body


@functools.lru_cache(maxsize=None)
def _make_sc_scatter(E, NP, DE, EW, KCH):
    FW = EW * DE
    NLOAD = FW // (KCH * 128)
    RW = (NP // NS) * DE

    @functools.partial(
        pl.kernel,
        out_type=jax.ShapeDtypeStruct((NC, NP * DE), F32),
        mesh=_mesh(),
        scratch_types=[
            pltpu.VMEM((KCH, 128), jnp.int32),
            pltpu.VMEM((KCH, 128), F32),
            pltpu.VMEM((RW,), F32),
            pltpu.VMEM_SHARED((NP * DE,), F32),
        ],
    )
    def body(msg3, fidx3, out, ibuf, mbuf, zbuf, agg_sh):
        c = lax.axis_index("c")
        s = lax.axis_index("s")
        wid = s * NC + c

        def zb(r, carry):
            zbuf[pl.ds(r * 16, 16)] = jnp.zeros((16,), F32)
            return carry

        lax.fori_loop(0, RW // 16, zb, 0, unroll=False)
        pltpu.sync_copy(zbuf, agg_sh.at[pl.ds(s * RW, RW)])
        plsc.subcore_barrier()

        def chunk(g, carry):
            pltpu.sync_copy(fidx3.at[wid * NLOAD + g], ibuf)
            pltpu.sync_copy(msg3.at[wid * NLOAD + g], mbuf)
            for k in range(KCH):
                pltpu.sync_copy(mbuf.at[k], agg_sh.at[ibuf.at[k]], add=True)
            return carry

        lax.fori_loop(0, NLOAD, chunk, 0, unroll=False)
        plsc.subcore_barrier()
        pltpu.sync_copy(agg_sh.at[pl.ds(s * RW, RW)], zbuf)
        pltpu.sync_copy(zbuf, out.at[c, pl.ds(s * RW, RW)])

    return body


def kernel(x, edge_index, edge_attr, num_nodes, W_e1, b_e1, W_e2, b_e2, W_n1, b_n1, W_n2, b_n2):
    del num_nodes
    N, D = x.shape
    E, DE = edge_attr.shape
    H = W_e1.shape[1]
    EW = E // NW
    NCHUNK = EW // CHUNK

    src = edge_index[0].astype(jnp.int32)
    dst = edge_index[1].astype(jnp.int32)
    src3 = src.reshape(NW, NCHUNK, CHUNK)
    dst3 = dst.reshape(NW, NCHUNK, CHUNK)

    RPT = -(-x.shape[0] // NS)
    RPT = -(-RPT // 128) * 128

    KCH = 25
    NLOAD = EW * DE // (KCH * 128)
    fidx3 = (dst[:, None] * DE + jnp.arange(DE, dtype=jnp.int32)[None, :]).reshape(
        NW * NLOAD, KCH, 128
    )

    W_src = W_e1[:D]
    W_dst = W_e1[D : 2 * D]
    W_ea = W_e1[2 * D :]
    W_n1x = W_n1[:D]
    W_n1a = W_n1[D:]
    b_e1r = b_e1.reshape(1, H)
    b_e2r = b_e2.reshape(1, DE)
    b_n1r = b_n1.reshape(1, H)
    b_n2r = b_n2.reshape(1, D)

    NP = RPT * NS

    proj = _make_proj(N, D, H, 2000)
    edge_tail = _make_edge_tail(E, DE, H, 4000)
    node = _make_node(N, NP, D, DE, H, 2000)
    sc_gather = _make_sc_gather(E, N, H, EW, NCHUNK)
    sc_scatter = _make_sc_scatter(E, NP, DE, EW, KCH)

    xcur = x
    msg = None
    for step in range(3):
        psrc, pdst = proj(xcur, W_src, W_dst)
        hpre = sc_gather(psrc, pdst, src3, dst3)
        msg = edge_tail(hpre, edge_attr, W_ea, b_e1r, W_e2, b_e2r)
        if step < 2:
            msg3 = msg.reshape(NW * NLOAD, KCH, 128)
            aggp = sc_scatter(msg3, fidx3).reshape(NC, NP, DE)
            xcur = node(xcur, aggp, W_n1x, W_n1a, b_n1r, W_n2, b_n2r)
    return (xcur, msg)


# speedup vs baseline: 2.1608x; 1.8658x over previous
import functools

import jax
import jax.numpy as jnp
from jax import lax
from jax.experimental import pallas as pl
from jax.experimental.pallas import tpu as pltpu
from jax.experimental.pallas import tpu_sc as plsc

F32 = jnp.float32
_PREC = lax.Precision.DEFAULT

NC = 2
NS = 16
NW = NC * NS
CHUNK = 40


def _mesh():
    return plsc.VectorSubcoreMesh(
        core_axis_name="c", subcore_axis_name="s", num_cores=NC, num_subcores=NS
    )


@functools.lru_cache(maxsize=None)
def _make_proj(N, D, H, MB):
    def body(x_ref, ws_ref, wd_ref, ps_ref, pd_ref):
        xv = x_ref[...]
        ps_ref[...] = jnp.dot(xv, ws_ref[...], precision=_PREC, preferred_element_type=F32)
        pd_ref[...] = jnp.dot(xv, wd_ref[...], precision=_PREC, preferred_element_type=F32)

    return pl.pallas_call(
        body,
        grid=(N // MB,),
        in_specs=[
            pl.BlockSpec((MB, D), lambda i: (i, 0)),
            pl.BlockSpec((D, H), lambda i: (0, 0)),
            pl.BlockSpec((D, H), lambda i: (0, 0)),
        ],
        out_specs=[
            pl.BlockSpec((MB, H), lambda i: (i, 0)),
            pl.BlockSpec((MB, H), lambda i: (i, 0)),
        ],
        out_shape=[
            jax.ShapeDtypeStruct((N, H), F32),
            jax.ShapeDtypeStruct((N, H), F32),
        ],
    )


@functools.lru_cache(maxsize=None)
def _make_edge_tail(E, DE, H, EB):
    def body(hp_ref, ea_ref, wea_ref, b1_ref, we2_ref, b2_ref, out_ref):
        h = hp_ref[...] + jnp.dot(
            ea_ref[...], wea_ref[...], precision=_PREC, preferred_element_type=F32
        ) + b1_ref[...]
        h = jnp.maximum(h, 0.0)
        out_ref[...] = jnp.dot(
            h, we2_ref[...], precision=_PREC, preferred_element_type=F32
        ) + b2_ref[...]

    return pl.pallas_call(
        body,
        grid=(E // EB,),
        in_specs=[
            pl.BlockSpec((EB, H), lambda i: (i, 0)),
            pl.BlockSpec((EB, DE), lambda i: (i, 0)),
            pl.BlockSpec((DE, H), lambda i: (0, 0)),
            pl.BlockSpec((1, H), lambda i: (0, 0)),
            pl.BlockSpec((H, DE), lambda i: (0, 0)),
            pl.BlockSpec((1, DE), lambda i: (0, 0)),
        ],
        out_specs=pl.BlockSpec((EB, DE), lambda i: (i, 0)),
        out_shape=jax.ShapeDtypeStruct((E, DE), F32),
    )


@functools.lru_cache(maxsize=None)
def _make_node(N, NP, D, DE, H, MB):
    def body(x_ref, agg_ref, w1x_ref, w1a_ref, b1_ref, w2_ref, b2_ref, out_ref):
        agg = agg_ref[0] + agg_ref[1]
        h = (
            jnp.dot(x_ref[...], w1x_ref[...], precision=_PREC, preferred_element_type=F32)
            + jnp.dot(agg, w1a_ref[...], precision=_PREC, preferred_element_type=F32)
            + b1_ref[...]
        )
        h = jnp.maximum(h, 0.0)
        out_ref[...] = jnp.dot(
            h, w2_ref[...], precision=_PREC, preferred_element_type=F32
        ) + b2_ref[...]

    return pl.pallas_call(
        body,
        grid=(N // MB,),
        in_specs=[
            pl.BlockSpec((MB, D), lambda i: (i, 0)),
            pl.BlockSpec((NC, MB, DE), lambda i: (0, i, 0)),
            pl.BlockSpec((D, H), lambda i: (0, 0)),
            pl.BlockSpec((DE, H), lambda i: (0, 0)),
            pl.BlockSpec((1, H), lambda i: (0, 0)),
            pl.BlockSpec((H, D), lambda i: (0, 0)),
            pl.BlockSpec((1, D), lambda i: (0, 0)),
        ],
        out_specs=pl.BlockSpec((MB, D), lambda i: (i, 0)),
        out_shape=jax.ShapeDtypeStruct((N, D), F32),
    )


@functools.lru_cache(maxsize=None)
def _make_sc_gather(E, N, H, EW, NCHUNK):
    assert NCHUNK % 2 == 1

    @functools.partial(
        pl.kernel,
        out_type=jax.ShapeDtypeStruct((E, H), F32),
        mesh=_mesh(),
        scratch_types=[
            pltpu.VMEM((NCHUNK, CHUNK), jnp.int32),
            pltpu.VMEM((NCHUNK, CHUNK), jnp.int32),
            pltpu.VMEM((CHUNK, H), F32),
            pltpu.VMEM((CHUNK, H), F32),
            pltpu.VMEM((CHUNK, H), F32),
            pltpu.VMEM((CHUNK, H), F32),
            pltpu.SemaphoreType.DMA,
            pltpu.SemaphoreType.DMA,
            pltpu.SemaphoreType.DMA,
            pltpu.SemaphoreType.DMA,
        ],
    )
    def body(psrc, pdst, src3, dst3, out, sidx, didx, bufa0, bufb0, bufa1, bufb1, sa0, sb0, sa1, sb1):
        c = lax.axis_index("c")
        s = lax.axis_index("s")
        wid = s * NC + c
        pltpu.sync_copy(src3.at[wid], sidx)
        pltpu.sync_copy(dst3.at[wid], didx)
        base = wid * EW

        def gather(j, bufa, bufb, sema, semb):
            return (
                pltpu.async_copy(psrc.at[sidx.at[j]], bufa, sema),
                pltpu.async_copy(pdst.at[didx.at[j]], bufb, semb),
            )

        def consume(j, bufa, bufb, sema, semb):
            pltpu.make_async_copy(psrc.at[sidx.at[j]], bufa, sema).wait()
            pltpu.make_async_copy(pdst.at[didx.at[j]], bufb, semb).wait()

            def row_body(r, carry2):
                for k in range(H // 16):
                    sl = pl.ds(k * 16, 16)
                    bufa[r, sl] = bufa[r, sl] + bufb[r, sl]
                return carry2

            lax.fori_loop(0, CHUNK, row_body, 0, unroll=False)
            pltpu.sync_copy(bufa, out.at[pl.ds(base + j * CHUNK, CHUNK)])

        gather(0, bufa0, bufb0, sa0, sb0)

        def pair_body(p, carry):
            gather(2 * p + 1, bufa1, bufb1, sa1, sb1)
            consume(2 * p, bufa0, bufb0, sa0, sb0)
            gather(2 * p + 2, bufa0, bufb0, sa0, sb0)
            consume(2 * p + 1, bufa1, bufb1, sa1, sb1)
            return carry

        lax.fori_loop(0, (NCHUNK - 1) // 2, pair_body, 0, unroll=False)
        consume(NCHUNK - 1, bufa0, bufb0, sa0, sb0)

    return body


@functools.lru_cache(maxsize=None)
def _make_sc_scatter(E, NP, DE, EW, KCH):
    FW = EW * DE
    NLOAD = FW // (KCH * 128)
    RW = (NP // NS) * DE

    @functools.partial(
        pl.kernel,
        out_type=jax.ShapeDtypeStruct((NC, NP * DE), F32),
        mesh=_mesh(),
        scratch_types=[
            pltpu.VMEM((KCH, 128), jnp.int32),
            pltpu.VMEM((KCH, 128), F32),
            pltpu.VMEM((RW,), F32),
            pltpu.VMEM_SHARED((NP * DE,), F32),
        ],
    )
    def body(msg3, fidx3, out, ibuf, mbuf, zbuf, agg_sh):
        c = lax.axis_index("c")
        s = lax.axis_index("s")
        wid = s * NC + c

        def zb(r, carry):
            zbuf[pl.ds(r * 16, 16)] = jnp.zeros((16,), F32)
            return carry

        lax.fori_loop(0, RW // 16, zb, 0, unroll=False)
        pltpu.sync_copy(zbuf, agg_sh.at[pl.ds(s * RW, RW)])
        plsc.subcore_barrier()

        def chunk(g, carry):
            pltpu.sync_copy(fidx3.at[wid * NLOAD + g], ibuf)
            pltpu.sync_copy(msg3.at[wid * NLOAD + g], mbuf)
            for k in range(KCH):
                pltpu.sync_copy(mbuf.at[k], agg_sh.at[ibuf.at[k]], add=True)
            return carry

        lax.fori_loop(0, NLOAD, chunk, 0, unroll=False)
        plsc.subcore_barrier()
        pltpu.sync_copy(agg_sh.at[pl.ds(s * RW, RW)], zbuf)
        pltpu.sync_copy(zbuf, out.at[c, pl.ds(s * RW, RW)])

    return body


def kernel(x, edge_index, edge_attr, num_nodes, W_e1, b_e1, W_e2, b_e2, W_n1, b_n1, W_n2, b_n2):
    del num_nodes
    N, D = x.shape
    E, DE = edge_attr.shape
    H = W_e1.shape[1]
    EW = E // NW
    NCHUNK = EW // CHUNK

    src = edge_index[0].astype(jnp.int32)
    dst = edge_index[1].astype(jnp.int32)
    src3 = src.reshape(NW, NCHUNK, CHUNK)
    dst3 = dst.reshape(NW, NCHUNK, CHUNK)

    RPT = -(-x.shape[0] // NS)
    RPT = -(-RPT // 128) * 128

    KCH = 25
    NLOAD = EW * DE // (KCH * 128)
    fidx3 = (dst[:, None] * DE + jnp.arange(DE, dtype=jnp.int32)[None, :]).reshape(
        NW * NLOAD, KCH, 128
    )

    W_src = W_e1[:D]
    W_dst = W_e1[D : 2 * D]
    W_ea = W_e1[2 * D :]
    W_n1x = W_n1[:D]
    W_n1a = W_n1[D:]
    b_e1r = b_e1.reshape(1, H)
    b_e2r = b_e2.reshape(1, DE)
    b_n1r = b_n1.reshape(1, H)
    b_n2r = b_n2.reshape(1, D)

    NP = RPT * NS

    proj = _make_proj(N, D, H, 2000)
    edge_tail = _make_edge_tail(E, DE, H, 4000)
    node = _make_node(N, NP, D, DE, H, 2000)
    sc_gather = _make_sc_gather(E, N, H, EW, NCHUNK)
    sc_scatter = _make_sc_scatter(E, NP, DE, EW, KCH)

    xcur = x
    msg = None
    for step in range(3):
        psrc, pdst = proj(xcur, W_src, W_dst)
        hpre = sc_gather(psrc, pdst, src3, dst3)
        msg = edge_tail(hpre, edge_attr, W_ea, b_e1r, W_e2, b_e2r)
        if step < 2:
            msg3 = msg.reshape(NW * NLOAD, KCH, 128)
            aggp = sc_scatter(msg3, fidx3).reshape(NC, NP, DE)
            xcur = node(xcur, aggp, W_n1x, W_n1a, b_n1r, W_n2, b_n2r)
    return (xcur, msg)


# speedup vs baseline: 2.2846x; 1.0573x over previous
import functools

import jax
import jax.numpy as jnp
from jax import lax
from jax.experimental import pallas as pl
from jax.experimental.pallas import tpu as pltpu
from jax.experimental.pallas import tpu_sc as plsc

F32 = jnp.float32
_PREC = lax.Precision.DEFAULT

NC = 2
NS = 16
NW = NC * NS
CHUNK = 40


def _mesh():
    return plsc.VectorSubcoreMesh(
        core_axis_name="c", subcore_axis_name="s", num_cores=NC, num_subcores=NS
    )


@functools.lru_cache(maxsize=None)
def _make_proj(N, D, H, MB):
    def body(x_ref, ws_ref, wd_ref, ps_ref, pd_ref):
        xv = x_ref[...]
        ps_ref[...] = jnp.dot(xv, ws_ref[...], precision=_PREC, preferred_element_type=F32)
        pd_ref[...] = jnp.dot(xv, wd_ref[...], precision=_PREC, preferred_element_type=F32)

    return pl.pallas_call(
        body,
        grid=(N // MB,),
        in_specs=[
            pl.BlockSpec((MB, D), lambda i: (i, 0)),
            pl.BlockSpec((D, H), lambda i: (0, 0)),
            pl.BlockSpec((D, H), lambda i: (0, 0)),
        ],
        out_specs=[
            pl.BlockSpec((MB, H), lambda i: (i, 0)),
            pl.BlockSpec((MB, H), lambda i: (i, 0)),
        ],
        out_shape=[
            jax.ShapeDtypeStruct((N, H), F32),
            jax.ShapeDtypeStruct((N, H), F32),
        ],
    )


@functools.lru_cache(maxsize=None)
def _make_edge_tail(E, DE, H, EB):
    def body(hp_ref, ea_ref, wea_ref, b1_ref, we2_ref, b2_ref, out_ref):
        h = hp_ref[...] + jnp.dot(
            ea_ref[...], wea_ref[...], precision=_PREC, preferred_element_type=F32
        ) + b1_ref[...]
        h = jnp.maximum(h, 0.0)
        out_ref[...] = jnp.dot(
            h, we2_ref[...], precision=_PREC, preferred_element_type=F32
        ) + b2_ref[...]

    return pl.pallas_call(
        body,
        grid=(E // EB,),
        in_specs=[
            pl.BlockSpec((EB, H), lambda i: (i, 0)),
            pl.BlockSpec((EB, DE), lambda i: (i, 0)),
            pl.BlockSpec((DE, H), lambda i: (0, 0)),
            pl.BlockSpec((1, H), lambda i: (0, 0)),
            pl.BlockSpec((H, DE), lambda i: (0, 0)),
            pl.BlockSpec((1, DE), lambda i: (0, 0)),
        ],
        out_specs=pl.BlockSpec((EB, DE), lambda i: (i, 0)),
        out_shape=jax.ShapeDtypeStruct((E, DE), F32),
    )


@functools.lru_cache(maxsize=None)
def _make_node(N, NP, D, DE, H, MB):
    def body(x_ref, agg_ref, w1x_ref, w1a_ref, b1_ref, w2_ref, b2_ref, out_ref):
        agg = agg_ref[0] + agg_ref[1]
        h = (
            jnp.dot(x_ref[...], w1x_ref[...], precision=_PREC, preferred_element_type=F32)
            + jnp.dot(agg, w1a_ref[...], precision=_PREC, preferred_element_type=F32)
            + b1_ref[...]
        )
        h = jnp.maximum(h, 0.0)
        out_ref[...] = jnp.dot(
            h, w2_ref[...], precision=_PREC, preferred_element_type=F32
        ) + b2_ref[...]

    return pl.pallas_call(
        body,
        grid=(N // MB,),
        in_specs=[
            pl.BlockSpec((MB, D), lambda i: (i, 0)),
            pl.BlockSpec((NC, MB, DE), lambda i: (0, i, 0)),
            pl.BlockSpec((D, H), lambda i: (0, 0)),
            pl.BlockSpec((DE, H), lambda i: (0, 0)),
            pl.BlockSpec((1, H), lambda i: (0, 0)),
            pl.BlockSpec((H, D), lambda i: (0, 0)),
            pl.BlockSpec((1, D), lambda i: (0, 0)),
        ],
        out_specs=pl.BlockSpec((MB, D), lambda i: (i, 0)),
        out_shape=jax.ShapeDtypeStruct((N, D), F32),
    )


@functools.lru_cache(maxsize=None)
def _make_sc_gather(E, N, H, EW, NCHUNK):
    assert NCHUNK % 2 == 1

    @functools.partial(
        pl.kernel,
        out_type=jax.ShapeDtypeStruct((E, H), F32),
        mesh=_mesh(),
        scratch_types=[
            pltpu.VMEM((NCHUNK, CHUNK), jnp.int32),
            pltpu.VMEM((NCHUNK, CHUNK), jnp.int32),
            pltpu.VMEM((CHUNK, H), F32),
            pltpu.VMEM((CHUNK, H), F32),
            pltpu.VMEM((CHUNK, H), F32),
            pltpu.VMEM((CHUNK, H), F32),
            pltpu.SemaphoreType.DMA,
            pltpu.SemaphoreType.DMA,
            pltpu.SemaphoreType.DMA,
            pltpu.SemaphoreType.DMA,
        ],
    )
    def body(psrc, pdst, src3, dst3, out, sidx, didx, bufa0, bufb0, bufa1, bufb1, sa0, sb0, sa1, sb1):
        c = lax.axis_index("c")
        s = lax.axis_index("s")
        wid = s * NC + c
        pltpu.sync_copy(src3.at[wid], sidx)
        pltpu.sync_copy(dst3.at[wid], didx)
        base = wid * EW

        def gather(j, bufa, bufb, sema, semb):
            return (
                pltpu.async_copy(psrc.at[sidx.at[j]], bufa, sema),
                pltpu.async_copy(pdst.at[didx.at[j]], bufb, semb),
            )

        def consume(j, bufa, bufb, sema, semb):
            pltpu.make_async_copy(psrc.at[sidx.at[j]], bufa, sema).wait()
            pltpu.make_async_copy(pdst.at[didx.at[j]], bufb, semb).wait()

            def row_body(r, carry2):
                for k in range(H // 16):
                    sl = pl.ds(k * 16, 16)
                    bufa[r, sl] = bufa[r, sl] + bufb[r, sl]
                return carry2

            lax.fori_loop(0, CHUNK, row_body, 0, unroll=False)
            pltpu.sync_copy(bufa, out.at[pl.ds(base + j * CHUNK, CHUNK)])

        gather(0, bufa0, bufb0, sa0, sb0)

        def pair_body(p, carry):
            gather(2 * p + 1, bufa1, bufb1, sa1, sb1)
            consume(2 * p, bufa0, bufb0, sa0, sb0)
            gather(2 * p + 2, bufa0, bufb0, sa0, sb0)
            consume(2 * p + 1, bufa1, bufb1, sa1, sb1)
            return carry

        lax.fori_loop(0, (NCHUNK - 1) // 2, pair_body, 0, unroll=False)
        consume(NCHUNK - 1, bufa0, bufb0, sa0, sb0)

    return body


@functools.lru_cache(maxsize=None)
def _make_sc_scatter(E, NP, DE, EW, KCH):
    FW = EW * DE
    NLOAD = FW // (KCH * 128)
    RW = (NP // NS) * DE

    @functools.partial(
        pl.kernel,
        out_type=jax.ShapeDtypeStruct((NC, NP * DE), F32),
        mesh=_mesh(),
        scratch_types=[
            pltpu.VMEM((KCH, 128), jnp.int32),
            pltpu.VMEM((KCH, 128), F32),
            pltpu.VMEM((RW,), F32),
            pltpu.VMEM_SHARED((NP * DE,), F32),
            pltpu.SemaphoreType.DMA,
        ],
    )
    def body(msg3, fidx3, out, ibuf, mbuf, zbuf, agg_sh, sem):
        c = lax.axis_index("c")
        s = lax.axis_index("s")
        wid = s * NC + c

        def zb(r, carry):
            zbuf[pl.ds(r * 16, 16)] = jnp.zeros((16,), F32)
            return carry

        lax.fori_loop(0, RW // 16, zb, 0, unroll=False)
        pltpu.sync_copy(zbuf, agg_sh.at[pl.ds(s * RW, RW)])
        plsc.subcore_barrier()

        def chunk(g, carry):
            pltpu.sync_copy(fidx3.at[wid * NLOAD + g], ibuf)
            pltpu.sync_copy(msg3.at[wid * NLOAD + g], mbuf)
            descs = [
                pltpu.async_copy(mbuf.at[k], agg_sh.at[ibuf.at[k]], sem, add=True)
                for k in range(KCH)
            ]
            for d in descs:
                d.wait()
            return carry

        lax.fori_loop(0, NLOAD, chunk, 0, unroll=False)
        plsc.subcore_barrier()
        pltpu.sync_copy(agg_sh.at[pl.ds(s * RW, RW)], zbuf)
        pltpu.sync_copy(zbuf, out.at[c, pl.ds(s * RW, RW)])

    return body


def kernel(x, edge_index, edge_attr, num_nodes, W_e1, b_e1, W_e2, b_e2, W_n1, b_n1, W_n2, b_n2):
    del num_nodes
    N, D = x.shape
    E, DE = edge_attr.shape
    H = W_e1.shape[1]
    EW = E // NW
    NCHUNK = EW // CHUNK

    src = edge_index[0].astype(jnp.int32)
    dst = edge_index[1].astype(jnp.int32)
    src3 = src.reshape(NW, NCHUNK, CHUNK)
    dst3 = dst.reshape(NW, NCHUNK, CHUNK)

    RPT = -(-x.shape[0] // NS)
    RPT = -(-RPT // 128) * 128

    KCH = 25
    NLOAD = EW * DE // (KCH * 128)
    fidx3 = (dst[:, None] * DE + jnp.arange(DE, dtype=jnp.int32)[None, :]).reshape(
        NW * NLOAD, KCH, 128
    )

    W_src = W_e1[:D]
    W_dst = W_e1[D : 2 * D]
    W_ea = W_e1[2 * D :]
    W_n1x = W_n1[:D]
    W_n1a = W_n1[D:]
    b_e1r = b_e1.reshape(1, H)
    b_e2r = b_e2.reshape(1, DE)
    b_n1r = b_n1.reshape(1, H)
    b_n2r = b_n2.reshape(1, D)

    NP = RPT * NS

    proj = _make_proj(N, D, H, 2000)
    edge_tail = _make_edge_tail(E, DE, H, 4000)
    node = _make_node(N, NP, D, DE, H, 2000)
    sc_gather = _make_sc_gather(E, N, H, EW, NCHUNK)
    sc_scatter = _make_sc_scatter(E, NP, DE, EW, KCH)

    xcur = x
    msg = None
    for step in range(3):
        psrc, pdst = proj(xcur, W_src, W_dst)
        hpre = sc_gather(psrc, pdst, src3, dst3)
        msg = edge_tail(hpre, edge_attr, W_ea, b_e1r, W_e2, b_e2r)
        if step < 2:
            msg3 = msg.reshape(NW * NLOAD, KCH, 128)
            aggp = sc_scatter(msg3, fidx3).reshape(NC, NP, DE)
            xcur = node(xcur, aggp, W_n1x, W_n1a, b_n1r, W_n2, b_n2r)
    return (xcur, msg)
